# Initial kernel scaffold; baseline (speedup 1.0000x reference)
import jax
import jax.numpy as jnp
from jax.experimental import pallas as pl


def kernel(x, pos, edge_index, edge_attr, emb_in_w, emb_in_b, edge_w1, edge_b1, edge_w2, edge_b2, att_w, att_b, coord_w1, coord_b1, coord_w2, node_w1, node_b1, node_w2, node_b2, emb_out_w, emb_out_b, head_w1, head_b1, head_w2, head_b2):
    raise NotImplementedError("write your pallas kernel here")


# speedup vs baseline: 1.8702x; 1.8702x over previous
import functools

import jax
import jax.numpy as jnp
from jax import lax
from jax.experimental import pallas as pl
from jax.experimental.pallas import tpu as pltpu
from jax.experimental.pallas import tpu_sc as plsc

N = 10000
E = 320000
D = 128
DE = 16
MX = 144
NLAYERS = 4
NC = 2
NS = 16
NW = NC * NS
LANES = 16
CHUNK = 128
NCHUNKS = E // CHUNK
KPW = -(-NCHUNKS // NW)
CHS = 32
NCHS = E // CHS
KPWS = -(-NCHS // NW)
SPAD = 10240
ROWS_PT = SPAD // NS
DRAIN = 128


def _sc_mesh():
    return plsc.VectorSubcoreMesh(core_axis_name="c", subcore_axis_name="s",
                                  num_cores=NC, num_subcores=NS)


def _mm(a, b):
    return jnp.dot(a, b, preferred_element_type=jnp.float32)


def _full(shape):
    nd = len(shape)
    return pl.BlockSpec(shape, lambda *_, _n=nd: (0,) * _n)


def _gather_layer(tr, tc, coord_flat, row, col):
    @functools.partial(
        pl.kernel,
        out_type=[jax.ShapeDtypeStruct((E, D), jnp.float32),
                  jax.ShapeDtypeStruct((E * 4,), jnp.float32)],
        mesh=_sc_mesh(),
        compiler_params=pltpu.CompilerParams(needs_layout_passes=False),
        scratch_types=[
            pltpu.VMEM((N * 4,), jnp.float32),
            pltpu.VMEM((CHUNK,), jnp.int32),
            pltpu.VMEM((CHUNK,), jnp.int32),
            pltpu.VMEM((CHUNK, D), jnp.float32),
            pltpu.VMEM((CHUNK, D), jnp.float32),
            pltpu.VMEM((CHUNK * 4,), jnp.float32),
            pltpu.SemaphoreType.DMA,
            pltpu.SemaphoreType.DMA,
        ],
    )
    def k(tr_hbm, tc_hbm, coord_hbm, row_hbm, col_hbm, z0_hbm, cd_hbm,
          ctab, ridx, cidx, rows_a, rows_b, cdc, sem_a, sem_b):
        cid = lax.axis_index("c")
        sid = lax.axis_index("s")
        wid = sid * NC + cid
        pltpu.sync_copy(coord_hbm, ctab)

        def chunk_body(kk, carry):
            t = wid + NW * kk

            @pl.when(t < NCHUNKS)
            def _do():
                base = t * CHUNK
                pltpu.sync_copy(row_hbm.at[pl.ds(base, CHUNK)], ridx)
                pltpu.sync_copy(col_hbm.at[pl.ds(base, CHUNK)], cidx)
                ca = pltpu.async_copy(tr_hbm.at[ridx], rows_a, sem_a)
                cb = pltpu.async_copy(tc_hbm.at[cidx], rows_b, sem_b)
                for g in range(CHUNK // LANES):
                    sl = pl.ds(g * LANES, LANES)
                    r4 = ridx[sl] * 4
                    c4 = cidx[sl] * 4
                    ii4 = (lax.iota(jnp.int32, LANES) + g * LANES) * 4
                    rad = jnp.zeros((LANES,), jnp.float32)
                    for kd in range(3):
                        dv = (plsc.load_gather(ctab, [r4 + kd])
                              - plsc.load_gather(ctab, [c4 + kd]))
                        plsc.store_scatter(cdc, [ii4 + kd], dv)
                        rad = rad + dv * dv
                    plsc.store_scatter(cdc, [ii4 + 3], rad)
                ca.wait()
                cb.wait()

                def add_row(i, c):
                    for j in range(D // LANES):
                        s2 = pl.ds(j * LANES, LANES)
                        rows_a[i, s2] = rows_a[i, s2] + rows_b[i, s2]
                    return c

                lax.fori_loop(0, CHUNK, add_row, 0)
                pltpu.sync_copy(rows_a, z0_hbm.at[pl.ds(base, CHUNK)])
                pltpu.sync_copy(cdc, cd_hbm.at[pl.ds(base * 4, CHUNK * 4)])

            return carry

        lax.fori_loop(0, KPW, chunk_body, 0)

    return k(tr, tc, coord_flat, row, col)


def _scatter_layer(m, s, cd, row):
    @functools.partial(
        pl.kernel,
        out_type=[jax.ShapeDtypeStruct((NC, SPAD, D), jnp.float32),
                  jax.ShapeDtypeStruct((NW, 4 * SPAD), jnp.float32)],
        mesh=_sc_mesh(),
        compiler_params=pltpu.CompilerParams(needs_layout_passes=False),
        scratch_types=[
            pltpu.VMEM((CHS, D), jnp.float32),
            pltpu.VMEM((CHS,), jnp.int32),
            pltpu.VMEM((CHS,), jnp.float32),
            pltpu.VMEM((CHS * 4,), jnp.float32),
            pltpu.VMEM((4 * SPAD,), jnp.float32),
            pltpu.VMEM_SHARED((SPAD, D), jnp.float32),
        ],
    )
    def k(m_hbm, s_hbm, cd_hbm, row_hbm, agg_hbm, tcn_hbm,
          mch, ridx, svec, cdc, tpart, agg_sp):
        cid = lax.axis_index("c")
        sid = lax.axis_index("s")
        wid = sid * NC + cid

        def zrow(i, c):
            for j in range(D // LANES):
                mch[i, pl.ds(j * LANES, LANES)] = jnp.zeros((LANES,),
                                                            jnp.float32)
            return c

        lax.fori_loop(0, CHS, zrow, 0)

        def ztc(i, c):
            tpart[pl.ds(i * LANES, LANES)] = jnp.zeros((LANES,), jnp.float32)
            return c

        lax.fori_loop(0, 4 * SPAD // LANES, ztc, 0)
        for c in range(ROWS_PT // CHS):
            st = sid * ROWS_PT + c * CHS
            pltpu.sync_copy(mch, agg_sp.at[pl.ds(st, CHS)])
        plsc.subcore_barrier()

        def chunk_body(kk, carry):
            t = wid + NW * kk

            @pl.when(t < NCHS)
            def _do():
                base = t * CHS
                pltpu.sync_copy(row_hbm.at[pl.ds(base, CHS)], ridx)
                pltpu.sync_copy(m_hbm.at[pl.ds(base, CHS)], mch)
                pltpu.sync_copy(s_hbm.at[pl.ds(base, CHS)], svec)
                pltpu.sync_copy(cd_hbm.at[pl.ds(base * 4, CHS * 4)], cdc)
                pltpu.sync_copy(mch, agg_sp.at[ridx], add=True)
                ones = jnp.ones((LANES,), jnp.float32)
                for g in range(CHS // LANES):
                    sl = pl.ds(g * LANES, LANES)
                    r4 = ridx[sl] * 4
                    s16 = svec[sl]
                    ii4 = (lax.iota(jnp.int32, LANES) + g * LANES) * 4
                    for kd in range(3):
                        cdv = plsc.load_gather(cdc, [ii4 + kd])
                        plsc.addupdate_scatter(tpart, [r4 + kd], cdv * s16)
                    plsc.addupdate_scatter(tpart, [r4 + 3], ones)

            return carry

        lax.fori_loop(0, KPWS, chunk_body, 0)
        plsc.subcore_barrier()

        for c in range(ROWS_PT // CHS):
            st = sid * ROWS_PT + c * CHS
            pltpu.sync_copy(agg_sp.at[pl.ds(st, CHS)], mch)
            pltpu.sync_copy(mch, agg_hbm.at[cid, pl.ds(st, CHS)])
        pltpu.sync_copy(tpart, tcn_hbm.at[wid])

    return k(m, s, cd, row)


NB = 1000


def _embed_prep(x, emb_w, emb_b, w1a, w1b, b1):
    def body(x_ref, ew, eb, wa, wb, b1r, h_ref, tr_ref, tc_ref):
        h = _mm(x_ref[...], ew[...]) + eb[...]
        h_ref[...] = h
        tr_ref[...] = _mm(h, wa[...]) + b1r[...]
        tc_ref[...] = _mm(h, wb[...])

    return pl.pallas_call(
        body, grid=(N // NB,),
        in_specs=[pl.BlockSpec((NB, D), lambda i: (i, 0)),
                  _full((D, D)), _full((1, D)), _full((D, D)), _full((D, D)),
                  _full((1, D))],
        out_specs=[pl.BlockSpec((NB, D), lambda i: (i, 0))] * 3,
        out_shape=[jax.ShapeDtypeStruct((N, D), jnp.float32)] * 3,
    )(x, emb_w, emb_b, w1a, w1b, b1)


def _edge_mlp(z0, cd, ea, radw, eaw, w2, b2, attw, attb, cw1, cb1, cw2):
    EB = 512

    def body(z0_ref, cd_ref, ea_ref, radw_r, eaw_r, w2_r, b2_r, attw_r,
             attb_r, cw1_r, cb1_r, cw2_r, m_ref, s_ref):
        rad = cd_ref[:, 3:4]
        z = z0_ref[...] + rad * radw_r[...] + _mm(ea_ref[...], eaw_r[...])
        m1 = jax.nn.silu(z)
        m2 = jax.nn.silu(_mm(m1, w2_r[...]) + b2_r[...])
        att = jax.nn.sigmoid(_mm(m2, attw_r[...]) + attb_r[...])
        m = m2 * att
        cm = jax.nn.silu(_mm(m, cw1_r[...]) + cb1_r[...])
        s_ref[...] = _mm(cm, cw2_r[...])
        m_ref[...] = m

    return pl.pallas_call(
        body, grid=(E // EB,),
        in_specs=[pl.BlockSpec((EB, D), lambda i: (i, 0)),
                  pl.BlockSpec((EB, 4), lambda i: (i, 0)),
                  pl.BlockSpec((EB, DE), lambda i: (i, 0)),
                  _full((1, D)), _full((DE, D)), _full((D, D)), _full((1, D)),
                  _full((D, 1)), _full((1, 1)), _full((D, D)), _full((1, D)),
                  _full((D, 1))],
        out_specs=[pl.BlockSpec((EB, D), lambda i: (i, 0)),
                   pl.BlockSpec((EB, 1), lambda i: (i, 0))],
        out_shape=[jax.ShapeDtypeStruct((E, D), jnp.float32),
                   jax.ShapeDtypeStruct((E, 1), jnp.float32)],
    )(z0, cd, ea, radw, eaw, w2, b2, attw, attb, cw1, cb1, cw2)


def _node_update(h, aggP, n1a, n1b, nb1, n2, nb2, w1a, w1b, b1):
    def body(h_ref, agg_ref, n1a_r, n1b_r, nb1_r, n2_r, nb2_r, wa, wb, b1r,
             h_out, tr_out, tc_out):
        agg = agg_ref[0] + agg_ref[1]
        o = jax.nn.silu(_mm(h_ref[...], n1a_r[...]) + _mm(agg, n1b_r[...])
                        + nb1_r[...])
        o = _mm(o, n2_r[...]) + nb2_r[...]
        hn = h_ref[...] + o
        h_out[...] = hn
        tr_out[...] = _mm(hn, wa[...]) + b1r[...]
        tc_out[...] = _mm(hn, wb[...])

    return pl.pallas_call(
        body, grid=(N // NB,),
        in_specs=[pl.BlockSpec((NB, D), lambda i: (i, 0)),
                  pl.BlockSpec((NC, NB, D), lambda i: (0, i, 0)),
                  _full((D, D)), _full((D, D)), _full((1, D)), _full((D, D)),
                  _full((1, D)), _full((D, D)), _full((D, D)), _full((1, D))],
        out_specs=[pl.BlockSpec((NB, D), lambda i: (i, 0))] * 3,
        out_shape=[jax.ShapeDtypeStruct((N, D), jnp.float32)] * 3,
    )(h, aggP, n1a, n1b, nb1, n2, nb2, w1a, w1b, b1)


def _node_final(h, aggP, n1a, n1b, nb1, n2, nb2, ew, eb, hw1, hb1, hw2, hb2):
    def body(h_ref, agg_ref, n1a_r, n1b_r, nb1_r, n2_r, nb2_r, ew_r, eb_r,
             hw1_r, hb1_r, hw2_r, hb2_r, out_ref):
        agg = agg_ref[0] + agg_ref[1]
        o = jax.nn.silu(_mm(h_ref[...], n1a_r[...]) + _mm(agg, n1b_r[...])
                        + nb1_r[...])
        o = _mm(o, n2_r[...]) + nb2_r[...]
        hn = h_ref[...] + o
        he = _mm(hn, ew_r[...]) + eb_r[...]
        z = jax.nn.relu(_mm(he, hw1_r[...]) + hb1_r[...])
        out_ref[...] = _mm(z, hw2_r[...]) + hb2_r[...]

    return pl.pallas_call(
        body, grid=(N // NB,),
        in_specs=[pl.BlockSpec((NB, D), lambda i: (i, 0)),
                  pl.BlockSpec((NC, NB, D), lambda i: (0, i, 0)),
                  _full((D, D)), _full((D, D)), _full((1, D)), _full((D, D)),
                  _full((1, D)), _full((D, D)), _full((1, D)), _full((D, D)),
                  _full((1, D)), _full((D, 1)), _full((1, 1))],
        out_specs=pl.BlockSpec((NB, 1), lambda i: (i, 0)),
        out_shape=jax.ShapeDtypeStruct((N, 1), jnp.float32),
    )(h, aggP, n1a, n1b, nb1, n2, nb2, ew, eb, hw1, hb1, hw2, hb2)


def _coord_update(coord, tcnP):
    def body(c_ref, t_ref, o_ref, acc):
        i = pl.program_id(0)

        @pl.when(i == 0)
        def _init():
            acc[...] = t_ref[0]

        @pl.when(i > 0)
        def _accum():
            acc[...] = acc[...] + t_ref[0]

        @pl.when(i == NW - 1)
        def _fin():
            t = acc[...][:N]
            cnt = jnp.maximum(t[:, 3:4], 1.0)
            upd = t / cnt
            msk = lax.broadcasted_iota(jnp.int32, (N, 4), 1) < 3
            o_ref[...] = c_ref[...] + jnp.where(msk, upd, 0.0)

    return pl.pallas_call(
        body, grid=(NW,),
        in_specs=[_full((N, 4)),
                  pl.BlockSpec((1, SPAD, 4), lambda i: (i, 0, 0))],
        out_specs=_full((N, 4)),
        out_shape=jax.ShapeDtypeStruct((N, 4), jnp.float32),
        scratch_shapes=[pltpu.VMEM((SPAD, 4), jnp.float32)],
    )(coord, tcnP)


def kernel(x, pos, edge_index, edge_attr, emb_in_w, emb_in_b, edge_w1,
           edge_b1, edge_w2, edge_b2, att_w, att_b, coord_w1, coord_b1,
           coord_w2, node_w1, node_b1, node_w2, node_b2, emb_out_w,
           emb_out_b, head_w1, head_b1, head_w2, head_b2):
    row = edge_index[0]
    col = edge_index[1]
    coord_flat = jnp.pad(pos, ((0, 0), (0, 1))).reshape(-1)

    h, tr, tc = _embed_prep(x, emb_in_w, emb_in_b.reshape(1, D),
                            edge_w1[0][:D], edge_w1[0][D:2 * D],
                            edge_b1[0].reshape(1, D))
    logits = None
    for i in range(NLAYERS):
        z0, cdf = _gather_layer(tr, tc, coord_flat, row, col)
        cd = cdf.reshape(E, 4)
        m, s = _edge_mlp(z0, cd, edge_attr,
                         edge_w1[i][2 * D:2 * D + 1], edge_w1[i][2 * D + 1:],
                         edge_w2[i], edge_b2[i].reshape(1, D),
                         att_w[i], att_b[i].reshape(1, 1),
                         coord_w1[i], coord_b1[i].reshape(1, D), coord_w2[i])
        aggP, tcnP = _scatter_layer(m, s.reshape(E), cdf, row)
        if i < NLAYERS - 1:
            coord_flat = _coord_update(coord_flat.reshape(N, 4),
                                       tcnP.reshape(NW, SPAD, 4)).reshape(-1)
            h, tr, tc = _node_update(h, aggP, node_w1[i][:D],
                                     node_w1[i][D:],
                                     node_b1[i].reshape(1, D), node_w2[i],
                                     node_b2[i].reshape(1, D),
                                     edge_w1[i + 1][:D],
                                     edge_w1[i + 1][D:2 * D],
                                     edge_b1[i + 1].reshape(1, D))
        else:
            logits = _node_final(h, aggP, node_w1[i][:D],
                                 node_w1[i][D:],
                                 node_b1[i].reshape(1, D), node_w2[i],
                                 node_b2[i].reshape(1, D),
                                 emb_out_w, emb_out_b.reshape(1, D),
                                 head_w1, head_b1.reshape(1, D),
                                 head_w2, head_b2.reshape(1, 1))
    return logits


# speedup vs baseline: 2.9140x; 1.5581x over previous
import functools

import jax
import jax.numpy as jnp
from jax import lax
from jax.experimental import pallas as pl
from jax.experimental.pallas import tpu as pltpu
from jax.experimental.pallas import tpu_sc as plsc

N = 10000
E = 320000
D = 128
DE = 16
MX = 144
NLAYERS = 4
NC = 2
NS = 16
NW = NC * NS
LANES = 16
CHUNK = 128
NCHUNKS = E // CHUNK
KPW = -(-NCHUNKS // NW)
CHS = 64
NCHS = E // CHS
KPWS = -(-NCHS // NW)
SPAD = 10240
NP8 = SPAD // 8
ROWS_PT = SPAD // NS
DRAIN = 128


def _sc_mesh():
    return plsc.VectorSubcoreMesh(core_axis_name="c", subcore_axis_name="s",
                                  num_cores=NC, num_subcores=NS)


def _mm(a, b):
    return jnp.dot(a, b, preferred_element_type=jnp.float32)


def _full(shape):
    nd = len(shape)
    return pl.BlockSpec(shape, lambda *_, _n=nd: (0,) * _n)


def _gather_layer(tr, tc, coord_flat, row, col):
    kpairs = -(-KPW // 2)
    ktot = 2 * kpairs

    @functools.partial(
        pl.kernel,
        out_type=[jax.ShapeDtypeStruct((E, D), jnp.float32),
                  jax.ShapeDtypeStruct((E * 4,), jnp.float32)],
        mesh=_sc_mesh(),
        compiler_params=pltpu.CompilerParams(needs_layout_passes=False),
        scratch_types=[
            pltpu.VMEM((N * 4,), jnp.float32),
            [pltpu.VMEM((CHUNK,), jnp.int32)] * 2,
            [pltpu.VMEM((CHUNK,), jnp.int32)] * 2,
            [pltpu.VMEM((CHUNK, D), jnp.float32)] * 2,
            [pltpu.VMEM((CHUNK, D), jnp.float32)] * 2,
            [pltpu.VMEM((CHUNK * 4,), jnp.float32)] * 2,
            [pltpu.SemaphoreType.DMA] * 2,
            [pltpu.SemaphoreType.DMA] * 2,
            [pltpu.SemaphoreType.DMA] * 2,
        ],
    )
    def k(tr_hbm, tc_hbm, coord_hbm, row_hbm, col_hbm, z0_hbm, cd_hbm,
          ctab, ridx, cidx, rows_a, rows_b, cdc, sem_i, sem_g, sem_s):
        cid = lax.axis_index("c")
        sid = lax.axis_index("s")
        wid = sid * NC + cid
        pltpu.sync_copy(coord_hbm, ctab)

        def tof(k_lin):
            return wid + NW * k_lin

        def valid(k_lin):
            return tof(k_lin) < NCHUNKS

        def issue_idx(k_lin, b):
            base = tof(k_lin) * CHUNK
            pltpu.async_copy(row_hbm.at[pl.ds(base, CHUNK)], ridx[b],
                             sem_i[b])
            pltpu.async_copy(col_hbm.at[pl.ds(base, CHUNK)], cidx[b],
                             sem_i[b])

        def wait_idx(b):
            pltpu.make_async_copy(row_hbm.at[pl.ds(0, CHUNK)], ridx[b],
                                  sem_i[b]).wait()
            pltpu.make_async_copy(col_hbm.at[pl.ds(0, CHUNK)], cidx[b],
                                  sem_i[b]).wait()

        def issue_gath(b):
            pltpu.async_copy(tr_hbm.at[ridx[b]], rows_a[b], sem_g[b])
            pltpu.async_copy(tc_hbm.at[cidx[b]], rows_b[b], sem_g[b])

        def wait_gath(b):
            pltpu.make_async_copy(tr_hbm.at[ridx[b]], rows_a[b],
                                  sem_g[b]).wait()
            pltpu.make_async_copy(tc_hbm.at[cidx[b]], rows_b[b],
                                  sem_g[b]).wait()

        def issue_store(k_lin, b):
            base = tof(k_lin) * CHUNK
            pltpu.async_copy(rows_a[b], z0_hbm.at[pl.ds(base, CHUNK)],
                             sem_s[b])
            pltpu.async_copy(cdc[b], cd_hbm.at[pl.ds(base * 4, CHUNK * 4)],
                             sem_s[b])

        def wait_store(b):
            pltpu.make_async_copy(rows_a[b], z0_hbm.at[pl.ds(0, CHUNK)],
                                  sem_s[b]).wait()
            pltpu.make_async_copy(cdc[b], cd_hbm.at[pl.ds(0, CHUNK * 4)],
                                  sem_s[b]).wait()

        def compute(k_lin, b):
            for g in range(CHUNK // LANES):
                sl = pl.ds(g * LANES, LANES)
                r4 = ridx[b][sl] * 4
                c4 = cidx[b][sl] * 4
                ii4 = (lax.iota(jnp.int32, LANES) + g * LANES) * 4
                rad = jnp.zeros((LANES,), jnp.float32)
                for kd in range(3):
                    dv = (plsc.load_gather(ctab, [r4 + kd])
                          - plsc.load_gather(ctab, [c4 + kd]))
                    plsc.store_scatter(cdc[b], [ii4 + kd], dv)
                    rad = rad + dv * dv
                plsc.store_scatter(cdc[b], [ii4 + 3], rad)

            def add_row(i, c):
                for j in range(D // LANES):
                    s2 = pl.ds(j * LANES, LANES)
                    rows_a[b][i, s2] = rows_a[b][i, s2] + rows_b[b][i, s2]
                return c

            lax.fori_loop(0, CHUNK, add_row, 0)
            issue_store(k_lin, b)

        @pl.when(valid(0))
        def _p0():
            issue_idx(0, 0)

        @pl.when(valid(1))
        def _p1():
            issue_idx(1, 1)

        @pl.when(valid(0))
        def _p2():
            wait_idx(0)
            issue_gath(0)

        def pair_body(kk, carry):
            for b in range(2):
                k_lin = 2 * kk + b
                nb = 1 - b

                @pl.when(valid(k_lin))
                def _s1():
                    wait_gath(b)

                @pl.when((k_lin >= 1) & valid(k_lin - 1))
                def _s2a():
                    wait_store(nb)

                @pl.when(valid(k_lin + 1))
                def _s2b():
                    wait_idx(nb)
                    issue_gath(nb)

                @pl.when(valid(k_lin))
                def _s4():
                    compute(k_lin, b)

                @pl.when(valid(k_lin + 2))
                def _s5():
                    issue_idx(k_lin + 2, b)

            return carry

        lax.fori_loop(0, kpairs, pair_body, 0)

        @pl.when(valid(ktot - 1))
        def _e1():
            wait_store((ktot - 1) % 2)

    return k(tr, tc, coord_flat, row, col)


def _scatter_layer(m, tq, row):
    kpairs = -(-KPWS // 2)
    ktot = 2 * kpairs

    @functools.partial(
        pl.kernel,
        out_type=[jax.ShapeDtypeStruct((NC, SPAD, D), jnp.float32),
                  jax.ShapeDtypeStruct((NC, NP8, D), jnp.float32)],
        mesh=_sc_mesh(),
        compiler_params=pltpu.CompilerParams(needs_layout_passes=False),
        scratch_types=[
            [pltpu.VMEM((CHS, D), jnp.float32)] * 2,
            [pltpu.VMEM((CHS,), jnp.int32)] * 2,
            [pltpu.VMEM((CHS,), jnp.int32)] * 2,
            [pltpu.VMEM((CHS * 8,), jnp.float32)] * 2,
            [pltpu.VMEM((CHS, D), jnp.float32)] * 2,
            [pltpu.VMEM((CHS,), jnp.int32)] * 2,
            [pltpu.SemaphoreType.DMA] * 2,
            [pltpu.SemaphoreType.DMA] * 2,
            pltpu.VMEM_SHARED((SPAD, D), jnp.float32),
            pltpu.VMEM_SHARED((NP8, D), jnp.float32),
        ],
    )
    def k(m_hbm, tq_hbm, row_hbm, agg_hbm, tcn_hbm,
          mch, ridx, r8, tqc, tbuf, opb, sem_l, sem_o, agg_sp, tcn_sp):
        cid = lax.axis_index("c")
        sid = lax.axis_index("s")
        wid = sid * NC + cid

        def tof(k_lin):
            return wid + NW * k_lin

        def valid(k_lin):
            return tof(k_lin) < NCHS

        def zrow(i, c):
            for j in range(D // LANES):
                z16 = jnp.zeros((LANES,), jnp.float32)
                mch[0][i, pl.ds(j * LANES, LANES)] = z16
                tbuf[0][i, pl.ds(j * LANES, LANES)] = z16
                tbuf[1][i, pl.ds(j * LANES, LANES)] = z16
            return c

        lax.fori_loop(0, CHS, zrow, 0)
        for g in range(CHS // LANES):
            sl = pl.ds(g * LANES, LANES)
            opb[0][sl] = jnp.zeros((LANES,), jnp.int32)
            opb[1][sl] = jnp.zeros((LANES,), jnp.int32)
        for c in range(ROWS_PT // CHS):
            st = sid * ROWS_PT + c * CHS
            pltpu.sync_copy(mch[0], agg_sp.at[pl.ds(st, CHS)])
        for c in range(2):
            st = sid * (NP8 // NS) + c * (NP8 // NS // 2)
            pltpu.sync_copy(mch[0].at[pl.ds(0, NP8 // NS // 2)],
                            tcn_sp.at[pl.ds(st, NP8 // NS // 2)])
        plsc.subcore_barrier()

        def issue_loads(k_lin, b):
            base = tof(k_lin) * CHS
            pltpu.async_copy(row_hbm.at[pl.ds(base, CHS)], ridx[b], sem_l[b])
            pltpu.async_copy(m_hbm.at[pl.ds(base, CHS)], mch[b], sem_l[b])
            pltpu.async_copy(tq_hbm.at[pl.ds(base * 8, CHS * 8)], tqc[b],
                             sem_l[b])

        def wait_loads(b):
            pltpu.make_async_copy(row_hbm.at[pl.ds(0, CHS)], ridx[b],
                                  sem_l[b]).wait()
            pltpu.make_async_copy(m_hbm.at[pl.ds(0, CHS)], mch[b],
                                  sem_l[b]).wait()
            pltpu.make_async_copy(tq_hbm.at[pl.ds(0, CHS * 8)], tqc[b],
                                  sem_l[b]).wait()

        def issue_scat(b):
            pltpu.async_copy(mch[b], agg_sp.at[ridx[b]], sem_o[b], add=True)
            pltpu.async_copy(tbuf[b], tcn_sp.at[r8[b]], sem_o[b], add=True)

        def wait_scat(b):
            pltpu.make_async_copy(mch[b], agg_sp.at[ridx[b]],
                                  sem_o[b]).wait()
            pltpu.make_async_copy(tbuf[b], tcn_sp.at[r8[b]],
                                  sem_o[b]).wait()

        def compute(b):
            z16 = jnp.zeros((LANES,), jnp.float32)
            for g in range(CHS // LANES):
                sl = pl.ds(g * LANES, LANES)
                r16 = ridx[b][sl]
                ii = lax.iota(jnp.int32, LANES) + g * LANES
                pbl = (r16 & 7) * LANES
                op = opb[b][sl]
                for kd in range(4):
                    plsc.store_scatter(tbuf[b], [ii, op + kd], z16)
                for kd in range(4):
                    tv = plsc.load_gather(tqc[b], [ii * 8 + kd])
                    plsc.store_scatter(tbuf[b], [ii, pbl + kd], tv)
                opb[b][sl] = pbl
                r8[b][sl] = r16 >> 3

        @pl.when(valid(0))
        def _p0():
            issue_loads(0, 0)

        def pair_body(kk, carry):
            for b in range(2):
                k_lin = 2 * kk + b
                nb = 1 - b

                @pl.when(valid(k_lin))
                def _s1():
                    wait_loads(b)

                @pl.when((k_lin >= 1) & valid(k_lin - 1))
                def _s2a():
                    wait_scat(nb)

                @pl.when(valid(k_lin + 1))
                def _s2b():
                    issue_loads(k_lin + 1, nb)

                @pl.when(valid(k_lin))
                def _s3():
                    compute(b)
                    issue_scat(b)

            return carry

        lax.fori_loop(0, kpairs, pair_body, 0)

        @pl.when(valid(ktot - 1))
        def _e1():
            wait_scat((ktot - 1) % 2)

        plsc.subcore_barrier()

        for c in range(ROWS_PT // CHS):
            st = sid * ROWS_PT + c * CHS
            pltpu.sync_copy(agg_sp.at[pl.ds(st, CHS)], mch[0])
            pltpu.sync_copy(mch[0], agg_hbm.at[cid, pl.ds(st, CHS)])
        for c in range(2):
            st = sid * (NP8 // NS) + c * (NP8 // NS // 2)
            pltpu.sync_copy(tcn_sp.at[pl.ds(st, NP8 // NS // 2)],
                            mch[0].at[pl.ds(0, NP8 // NS // 2)])
            pltpu.sync_copy(mch[0].at[pl.ds(0, NP8 // NS // 2)],
                            tcn_hbm.at[cid, pl.ds(st, NP8 // NS // 2)])

    return k(m, tq, row)


NB = 1000


def _embed_prep(x, emb_w, emb_b, w1a, w1b, b1):
    def body(x_ref, ew, eb, wa, wb, b1r, h_ref, tr_ref, tc_ref):
        h = _mm(x_ref[...], ew[...]) + eb[...]
        h_ref[...] = h
        tr_ref[...] = _mm(h, wa[...]) + b1r[...]
        tc_ref[...] = _mm(h, wb[...])

    return pl.pallas_call(
        body, grid=(N // NB,),
        in_specs=[pl.BlockSpec((NB, D), lambda i: (i, 0)),
                  _full((D, D)), _full((1, D)), _full((D, D)), _full((D, D)),
                  _full((1, D))],
        out_specs=[pl.BlockSpec((NB, D), lambda i: (i, 0))] * 3,
        out_shape=[jax.ShapeDtypeStruct((N, D), jnp.float32)] * 3,
    )(x, emb_w, emb_b, w1a, w1b, b1)


def _edge_mlp(z0, cd, ea, radw, eaw, w2, b2, attw, attb, cw1, cb1, cw2):
    EB = 512

    def body(z0_ref, cd_ref, ea_ref, radw_r, eaw_r, w2_r, b2_r, attw_r,
             attb_r, cw1_r, cb1_r, cw2_r, m_ref, tq_ref):
        cdb = cd_ref[...]
        rad = cdb[:, 3:4]
        z = z0_ref[...] + rad * radw_r[...] + _mm(ea_ref[...], eaw_r[...])
        m1 = jax.nn.silu(z)
        m2 = jax.nn.silu(_mm(m1, w2_r[...]) + b2_r[...])
        att = jax.nn.sigmoid(_mm(m2, attw_r[...]) + attb_r[...])
        m = m2 * att
        cm = jax.nn.silu(_mm(m, cw1_r[...]) + cb1_r[...])
        s = _mm(cm, cw2_r[...])
        m_ref[...] = m
        tq_ref[...] = jnp.concatenate(
            [cdb[:, 0:3] * s, jnp.ones((EB, 1), jnp.float32),
             jnp.zeros((EB, 4), jnp.float32)], axis=1)

    return pl.pallas_call(
        body, grid=(E // EB,),
        in_specs=[pl.BlockSpec((EB, D), lambda i: (i, 0)),
                  pl.BlockSpec((EB, 4), lambda i: (i, 0)),
                  pl.BlockSpec((EB, DE), lambda i: (i, 0)),
                  _full((1, D)), _full((DE, D)), _full((D, D)), _full((1, D)),
                  _full((D, 1)), _full((1, 1)), _full((D, D)), _full((1, D)),
                  _full((D, 1))],
        out_specs=[pl.BlockSpec((EB, D), lambda i: (i, 0)),
                   pl.BlockSpec((EB, 8), lambda i: (i, 0))],
        out_shape=[jax.ShapeDtypeStruct((E, D), jnp.float32),
                   jax.ShapeDtypeStruct((E, 8), jnp.float32)],
    )(z0, cd, ea, radw, eaw, w2, b2, attw, attb, cw1, cb1, cw2)


def _node_update(h, aggP, n1a, n1b, nb1, n2, nb2, w1a, w1b, b1):
    def body(h_ref, agg_ref, n1a_r, n1b_r, nb1_r, n2_r, nb2_r, wa, wb, b1r,
             h_out, tr_out, tc_out):
        agg = agg_ref[0] + agg_ref[1]
        o = jax.nn.silu(_mm(h_ref[...], n1a_r[...]) + _mm(agg, n1b_r[...])
                        + nb1_r[...])
        o = _mm(o, n2_r[...]) + nb2_r[...]
        hn = h_ref[...] + o
        h_out[...] = hn
        tr_out[...] = _mm(hn, wa[...]) + b1r[...]
        tc_out[...] = _mm(hn, wb[...])

    return pl.pallas_call(
        body, grid=(N // NB,),
        in_specs=[pl.BlockSpec((NB, D), lambda i: (i, 0)),
                  pl.BlockSpec((NC, NB, D), lambda i: (0, i, 0)),
                  _full((D, D)), _full((D, D)), _full((1, D)), _full((D, D)),
                  _full((1, D)), _full((D, D)), _full((D, D)), _full((1, D))],
        out_specs=[pl.BlockSpec((NB, D), lambda i: (i, 0))] * 3,
        out_shape=[jax.ShapeDtypeStruct((N, D), jnp.float32)] * 3,
    )(h, aggP, n1a, n1b, nb1, n2, nb2, w1a, w1b, b1)


def _node_final(h, aggP, n1a, n1b, nb1, n2, nb2, ew, eb, hw1, hb1, hw2, hb2):
    def body(h_ref, agg_ref, n1a_r, n1b_r, nb1_r, n2_r, nb2_r, ew_r, eb_r,
             hw1_r, hb1_r, hw2_r, hb2_r, out_ref):
        agg = agg_ref[0] + agg_ref[1]
        o = jax.nn.silu(_mm(h_ref[...], n1a_r[...]) + _mm(agg, n1b_r[...])
                        + nb1_r[...])
        o = _mm(o, n2_r[...]) + nb2_r[...]
        hn = h_ref[...] + o
        he = _mm(hn, ew_r[...]) + eb_r[...]
        z = jax.nn.relu(_mm(he, hw1_r[...]) + hb1_r[...])
        out_ref[...] = _mm(z, hw2_r[...]) + hb2_r[...]

    return pl.pallas_call(
        body, grid=(N // NB,),
        in_specs=[pl.BlockSpec((NB, D), lambda i: (i, 0)),
                  pl.BlockSpec((NC, NB, D), lambda i: (0, i, 0)),
                  _full((D, D)), _full((D, D)), _full((1, D)), _full((D, D)),
                  _full((1, D)), _full((D, D)), _full((1, D)), _full((D, D)),
                  _full((1, D)), _full((D, 1)), _full((1, 1))],
        out_specs=pl.BlockSpec((NB, 1), lambda i: (i, 0)),
        out_shape=jax.ShapeDtypeStruct((N, 1), jnp.float32),
    )(h, aggP, n1a, n1b, nb1, n2, nb2, ew, eb, hw1, hb1, hw2, hb2)


def _coord_update(coord, tcnP):
    def body(c_ref, t_ref, o_ref):
        t = t_ref[0, :N, 0:4] + t_ref[1, :N, 0:4]
        cnt = jnp.maximum(t[:, 3:4], 1.0)
        upd = t / cnt
        msk = lax.broadcasted_iota(jnp.int32, (N, 4), 1) < 3
        o_ref[...] = c_ref[...] + jnp.where(msk, upd, 0.0)

    return pl.pallas_call(
        body,
        in_specs=[_full((N, 4)), _full((NC, SPAD, 16))],
        out_specs=_full((N, 4)),
        out_shape=jax.ShapeDtypeStruct((N, 4), jnp.float32),
    )(coord, tcnP)


def kernel(x, pos, edge_index, edge_attr, emb_in_w, emb_in_b, edge_w1,
           edge_b1, edge_w2, edge_b2, att_w, att_b, coord_w1, coord_b1,
           coord_w2, node_w1, node_b1, node_w2, node_b2, emb_out_w,
           emb_out_b, head_w1, head_b1, head_w2, head_b2):
    row = edge_index[0]
    col = edge_index[1]
    coord_flat = jnp.pad(pos, ((0, 0), (0, 1))).reshape(-1)

    h, tr, tc = _embed_prep(x, emb_in_w, emb_in_b.reshape(1, D),
                            edge_w1[0][:D], edge_w1[0][D:2 * D],
                            edge_b1[0].reshape(1, D))
    logits = None
    for i in range(NLAYERS):
        z0, cdf = _gather_layer(tr, tc, coord_flat, row, col)
        cd = cdf.reshape(E, 4)
        m, tq = _edge_mlp(z0, cd, edge_attr,
                          edge_w1[i][2 * D:2 * D + 1], edge_w1[i][2 * D + 1:],
                          edge_w2[i], edge_b2[i].reshape(1, D),
                          att_w[i], att_b[i].reshape(1, 1),
                          coord_w1[i], coord_b1[i].reshape(1, D), coord_w2[i])
        aggP, tcnP = _scatter_layer(m, tq.reshape(-1), row)
        if i < NLAYERS - 1:
            coord_flat = _coord_update(coord_flat.reshape(N, 4),
                                       tcnP.reshape(NC, SPAD, 16)).reshape(-1)
            h, tr, tc = _node_update(h, aggP, node_w1[i][:D],
                                     node_w1[i][D:],
                                     node_b1[i].reshape(1, D), node_w2[i],
                                     node_b2[i].reshape(1, D),
                                     edge_w1[i + 1][:D],
                                     edge_w1[i + 1][D:2 * D],
                                     edge_b1[i + 1].reshape(1, D))
        else:
            logits = _node_final(h, aggP, node_w1[i][:D],
                                 node_w1[i][D:],
                                 node_b1[i].reshape(1, D), node_w2[i],
                                 node_b2[i].reshape(1, D),
                                 emb_out_w, emb_out_b.reshape(1, D),
                                 head_w1, head_b1.reshape(1, D),
                                 head_w2, head_b2.reshape(1, 1))
    return logits


# speedup vs baseline: 3.4953x; 1.1995x over previous
import functools

import jax
import jax.numpy as jnp
from jax import lax
from jax.experimental import pallas as pl
from jax.experimental.pallas import tpu as pltpu
from jax.experimental.pallas import tpu_sc as plsc

N = 10000
E = 320000
D = 128
DE = 16
MX = 144
NLAYERS = 4
NC = 2
NS = 16
NW = NC * NS
LANES = 16
CHUNK = 128
NCHUNKS = E // CHUNK
KPW = -(-NCHUNKS // NW)
CHS = 64
NCHS = E // CHS
KPWS = -(-NCHS // NW)
SPAD = 10240
NP8 = SPAD // 8
ROWS_PT = SPAD // NS
DRAIN = 128


def _sc_mesh():
    return plsc.VectorSubcoreMesh(core_axis_name="c", subcore_axis_name="s",
                                  num_cores=NC, num_subcores=NS)


def _mm(a, b):
    return jnp.dot(a, b, preferred_element_type=jnp.float32)


def _full(shape):
    nd = len(shape)
    return pl.BlockSpec(shape, lambda *_, _n=nd: (0,) * _n)


def _gather_layer(tr, tc, coord_flat, row, col):
    ne = row.shape[0]
    nchunks = ne // CHUNK
    kpw = -(-nchunks // NW)
    kpairs = -(-kpw // 2)
    ktot = 2 * kpairs

    @functools.partial(
        pl.kernel,
        out_type=[jax.ShapeDtypeStruct((ne, D), jnp.float32),
                  jax.ShapeDtypeStruct((ne * 4,), jnp.float32)],
        mesh=_sc_mesh(),
        compiler_params=pltpu.CompilerParams(needs_layout_passes=False),
        scratch_types=[
            pltpu.VMEM((N * 4,), jnp.float32),
            [pltpu.VMEM((CHUNK,), jnp.int32)] * 2,
            [pltpu.VMEM((CHUNK,), jnp.int32)] * 2,
            [pltpu.VMEM((CHUNK, D), jnp.float32)] * 2,
            [pltpu.VMEM((CHUNK, D), jnp.float32)] * 2,
            [pltpu.VMEM((CHUNK * 4,), jnp.float32)] * 2,
            [pltpu.SemaphoreType.DMA] * 2,
            [pltpu.SemaphoreType.DMA] * 2,
            [pltpu.SemaphoreType.DMA] * 2,
        ],
    )
    def k(tr_hbm, tc_hbm, coord_hbm, row_hbm, col_hbm, z0_hbm, cd_hbm,
          ctab, ridx, cidx, rows_a, rows_b, cdc, sem_i, sem_g, sem_s):
        cid = lax.axis_index("c")
        sid = lax.axis_index("s")
        wid = sid * NC + cid
        pltpu.sync_copy(coord_hbm, ctab)

        def tof(k_lin):
            return wid + NW * k_lin

        def valid(k_lin):
            return tof(k_lin) < nchunks

        def issue_idx(k_lin, b):
            base = tof(k_lin) * CHUNK
            pltpu.async_copy(row_hbm.at[pl.ds(base, CHUNK)], ridx[b],
                             sem_i[b])
            pltpu.async_copy(col_hbm.at[pl.ds(base, CHUNK)], cidx[b],
                             sem_i[b])

        def wait_idx(b):
            pltpu.make_async_copy(row_hbm.at[pl.ds(0, CHUNK)], ridx[b],
                                  sem_i[b]).wait()
            pltpu.make_async_copy(col_hbm.at[pl.ds(0, CHUNK)], cidx[b],
                                  sem_i[b]).wait()

        def issue_gath(b):
            pltpu.async_copy(tr_hbm.at[ridx[b]], rows_a[b], sem_g[b])
            pltpu.async_copy(tc_hbm.at[cidx[b]], rows_b[b], sem_g[b])

        def wait_gath(b):
            pltpu.make_async_copy(tr_hbm.at[ridx[b]], rows_a[b],
                                  sem_g[b]).wait()
            pltpu.make_async_copy(tc_hbm.at[cidx[b]], rows_b[b],
                                  sem_g[b]).wait()

        def issue_store(k_lin, b):
            base = tof(k_lin) * CHUNK
            pltpu.async_copy(rows_a[b], z0_hbm.at[pl.ds(base, CHUNK)],
                             sem_s[b])
            pltpu.async_copy(cdc[b], cd_hbm.at[pl.ds(base * 4, CHUNK * 4)],
                             sem_s[b])

        def wait_store(b):
            pltpu.make_async_copy(rows_a[b], z0_hbm.at[pl.ds(0, CHUNK)],
                                  sem_s[b]).wait()
            pltpu.make_async_copy(cdc[b], cd_hbm.at[pl.ds(0, CHUNK * 4)],
                                  sem_s[b]).wait()

        def compute(k_lin, b):
            for g in range(CHUNK // LANES):
                sl = pl.ds(g * LANES, LANES)
                r4 = ridx[b][sl] * 4
                c4 = cidx[b][sl] * 4
                ii4 = (lax.iota(jnp.int32, LANES) + g * LANES) * 4
                rad = jnp.zeros((LANES,), jnp.float32)
                for kd in range(3):
                    dv = (plsc.load_gather(ctab, [r4 + kd])
                          - plsc.load_gather(ctab, [c4 + kd]))
                    plsc.store_scatter(cdc[b], [ii4 + kd], dv)
                    rad = rad + dv * dv
                plsc.store_scatter(cdc[b], [ii4 + 3], rad)

            def add_row(i, c):
                for j in range(D // LANES):
                    s2 = pl.ds(j * LANES, LANES)
                    rows_a[b][i, s2] = rows_a[b][i, s2] + rows_b[b][i, s2]
                return c

            lax.fori_loop(0, CHUNK, add_row, 0)
            issue_store(k_lin, b)

        @pl.when(valid(0))
        def _p0():
            issue_idx(0, 0)

        @pl.when(valid(1))
        def _p1():
            issue_idx(1, 1)

        @pl.when(valid(0))
        def _p2():
            wait_idx(0)
            issue_gath(0)

        def pair_body(kk, carry):
            for b in range(2):
                k_lin = 2 * kk + b
                nb = 1 - b

                @pl.when(valid(k_lin))
                def _s1():
                    wait_gath(b)

                @pl.when((k_lin >= 1) & valid(k_lin - 1))
                def _s2a():
                    wait_store(nb)

                @pl.when(valid(k_lin + 1))
                def _s2b():
                    wait_idx(nb)
                    issue_gath(nb)

                @pl.when(valid(k_lin))
                def _s4():
                    compute(k_lin, b)

                @pl.when(valid(k_lin + 2))
                def _s5():
                    issue_idx(k_lin + 2, b)

            return carry

        lax.fori_loop(0, kpairs, pair_body, 0)

        @pl.when(valid(ktot - 1))
        def _e1():
            wait_store((ktot - 1) % 2)

    return k(tr, tc, coord_flat, row, col)


def _scatter_layer(m, tq, row):
    ne = row.shape[0]
    nchs = ne // CHS
    kpws = -(-nchs // NW)
    kpairs = -(-kpws // 2)
    ktot = 2 * kpairs

    @functools.partial(
        pl.kernel,
        out_type=[jax.ShapeDtypeStruct((NC, SPAD, D), jnp.float32),
                  jax.ShapeDtypeStruct((NC, NP8, D), jnp.float32)],
        mesh=_sc_mesh(),
        compiler_params=pltpu.CompilerParams(needs_layout_passes=False),
        scratch_types=[
            [pltpu.VMEM((CHS, D), jnp.float32)] * 2,
            [pltpu.VMEM((CHS,), jnp.int32)] * 2,
            [pltpu.VMEM((CHS,), jnp.int32)] * 2,
            [pltpu.VMEM((CHS * 8,), jnp.float32)] * 2,
            [pltpu.VMEM((CHS, D), jnp.float32)] * 2,
            [pltpu.VMEM((CHS,), jnp.int32)] * 2,
            [pltpu.SemaphoreType.DMA] * 2,
            [pltpu.SemaphoreType.DMA] * 2,
            pltpu.VMEM_SHARED((SPAD, D), jnp.float32),
            pltpu.VMEM_SHARED((NP8, D), jnp.float32),
        ],
    )
    def k(m_hbm, tq_hbm, row_hbm, agg_hbm, tcn_hbm,
          mch, ridx, r8, tqc, tbuf, opb, sem_l, sem_o, agg_sp, tcn_sp):
        cid = lax.axis_index("c")
        sid = lax.axis_index("s")
        wid = sid * NC + cid

        def tof(k_lin):
            return wid + NW * k_lin

        def valid(k_lin):
            return tof(k_lin) < nchs

        def zrow(i, c):
            for j in range(D // LANES):
                z16 = jnp.zeros((LANES,), jnp.float32)
                mch[0][i, pl.ds(j * LANES, LANES)] = z16
                tbuf[0][i, pl.ds(j * LANES, LANES)] = z16
                tbuf[1][i, pl.ds(j * LANES, LANES)] = z16
            return c

        lax.fori_loop(0, CHS, zrow, 0)
        for g in range(CHS // LANES):
            sl = pl.ds(g * LANES, LANES)
            opb[0][sl] = jnp.zeros((LANES,), jnp.int32)
            opb[1][sl] = jnp.zeros((LANES,), jnp.int32)
        for c in range(ROWS_PT // CHS):
            st = sid * ROWS_PT + c * CHS
            pltpu.sync_copy(mch[0], agg_sp.at[pl.ds(st, CHS)])
        for c in range(2):
            st = sid * (NP8 // NS) + c * (NP8 // NS // 2)
            pltpu.sync_copy(mch[0].at[pl.ds(0, NP8 // NS // 2)],
                            tcn_sp.at[pl.ds(st, NP8 // NS // 2)])
        plsc.subcore_barrier()

        def issue_loads(k_lin, b):
            base = tof(k_lin) * CHS
            pltpu.async_copy(row_hbm.at[pl.ds(base, CHS)], ridx[b], sem_l[b])
            pltpu.async_copy(m_hbm.at[pl.ds(base, CHS)], mch[b], sem_l[b])
            pltpu.async_copy(tq_hbm.at[pl.ds(base * 8, CHS * 8)], tqc[b],
                             sem_l[b])

        def wait_loads(b):
            pltpu.make_async_copy(row_hbm.at[pl.ds(0, CHS)], ridx[b],
                                  sem_l[b]).wait()
            pltpu.make_async_copy(m_hbm.at[pl.ds(0, CHS)], mch[b],
                                  sem_l[b]).wait()
            pltpu.make_async_copy(tq_hbm.at[pl.ds(0, CHS * 8)], tqc[b],
                                  sem_l[b]).wait()

        def issue_scat(b):
            pltpu.async_copy(mch[b], agg_sp.at[ridx[b]], sem_o[b], add=True)
            pltpu.async_copy(tbuf[b], tcn_sp.at[r8[b]], sem_o[b], add=True)

        def wait_scat(b):
            pltpu.make_async_copy(mch[b], agg_sp.at[ridx[b]],
                                  sem_o[b]).wait()
            pltpu.make_async_copy(tbuf[b], tcn_sp.at[r8[b]],
                                  sem_o[b]).wait()

        def compute(b):
            z16 = jnp.zeros((LANES,), jnp.float32)
            for g in range(CHS // LANES):
                sl = pl.ds(g * LANES, LANES)
                r16 = ridx[b][sl]
                ii = lax.iota(jnp.int32, LANES) + g * LANES
                pbl = (r16 & 7) * LANES
                op = opb[b][sl]
                for kd in range(4):
                    plsc.store_scatter(tbuf[b], [ii, op + kd], z16)
                for kd in range(4):
                    tv = plsc.load_gather(tqc[b], [ii * 8 + kd])
                    plsc.store_scatter(tbuf[b], [ii, pbl + kd], tv)
                opb[b][sl] = pbl
                r8[b][sl] = r16 >> 3

        @pl.when(valid(0))
        def _p0():
            issue_loads(0, 0)

        def pair_body(kk, carry):
            for b in range(2):
                k_lin = 2 * kk + b
                nb = 1 - b

                @pl.when(valid(k_lin))
                def _s1():
                    wait_loads(b)

                @pl.when((k_lin >= 1) & valid(k_lin - 1))
                def _s2a():
                    wait_scat(nb)

                @pl.when(valid(k_lin + 1))
                def _s2b():
                    issue_loads(k_lin + 1, nb)

                @pl.when(valid(k_lin))
                def _s3():
                    compute(b)
                    issue_scat(b)

            return carry

        lax.fori_loop(0, kpairs, pair_body, 0)

        @pl.when(valid(ktot - 1))
        def _e1():
            wait_scat((ktot - 1) % 2)

        plsc.subcore_barrier()

        for c in range(ROWS_PT // CHS):
            st = sid * ROWS_PT + c * CHS
            pltpu.sync_copy(agg_sp.at[pl.ds(st, CHS)], mch[0])
            pltpu.sync_copy(mch[0], agg_hbm.at[cid, pl.ds(st, CHS)])
        for c in range(2):
            st = sid * (NP8 // NS) + c * (NP8 // NS // 2)
            pltpu.sync_copy(tcn_sp.at[pl.ds(st, NP8 // NS // 2)],
                            mch[0].at[pl.ds(0, NP8 // NS // 2)])
            pltpu.sync_copy(mch[0].at[pl.ds(0, NP8 // NS // 2)],
                            tcn_hbm.at[cid, pl.ds(st, NP8 // NS // 2)])

    return k(m, tq, row)


NB = 1000


def _embed_prep(x, emb_w, emb_b, w1a, w1b, b1):
    def body(x_ref, ew, eb, wa, wb, b1r, h_ref, tr_ref, tc_ref):
        h = _mm(x_ref[...], ew[...]) + eb[...]
        h_ref[...] = h
        tr_ref[...] = _mm(h, wa[...]) + b1r[...]
        tc_ref[...] = _mm(h, wb[...])

    return pl.pallas_call(
        body, grid=(N // NB,),
        in_specs=[pl.BlockSpec((NB, D), lambda i: (i, 0)),
                  _full((D, D)), _full((1, D)), _full((D, D)), _full((D, D)),
                  _full((1, D))],
        out_specs=[pl.BlockSpec((NB, D), lambda i: (i, 0))] * 3,
        out_shape=[jax.ShapeDtypeStruct((N, D), jnp.float32)] * 3,
    )(x, emb_w, emb_b, w1a, w1b, b1)


def _edge_mlp(z0, cd, ea, radw, eaw, w2, b2, attw, attb, cw1, cb1, cw2):
    EB = 640

    def body(z0_ref, cd_ref, ea_ref, radw_r, eaw_r, w2_r, b2_r, attw_r,
             attb_r, cw1_r, cb1_r, cw2_r, m_ref, tq_ref):
        cdb = cd_ref[...]
        rad = cdb[:, 3:4]
        z = z0_ref[...] + rad * radw_r[...] + _mm(ea_ref[...], eaw_r[...])
        m1 = jax.nn.silu(z)
        m2 = jax.nn.silu(_mm(m1, w2_r[...]) + b2_r[...])
        att = jax.nn.sigmoid(_mm(m2, attw_r[...]) + attb_r[...])
        m = m2 * att
        cm = jax.nn.silu(_mm(m, cw1_r[...]) + cb1_r[...])
        s = _mm(cm, cw2_r[...])
        m_ref[...] = m
        tq_ref[...] = jnp.concatenate(
            [cdb[:, 0:3] * s, jnp.ones((EB, 1), jnp.float32),
             jnp.zeros((EB, 4), jnp.float32)], axis=1)

    return pl.pallas_call(
        body, grid=(z0.shape[0] // EB,),
        in_specs=[pl.BlockSpec((EB, D), lambda i: (i, 0)),
                  pl.BlockSpec((EB, 4), lambda i: (i, 0)),
                  pl.BlockSpec((EB, DE), lambda i: (i, 0)),
                  _full((1, D)), _full((DE, D)), _full((D, D)), _full((1, D)),
                  _full((D, 1)), _full((1, 1)), _full((D, D)), _full((1, D)),
                  _full((D, 1))],
        out_specs=[pl.BlockSpec((EB, D), lambda i: (i, 0)),
                   pl.BlockSpec((EB, 8), lambda i: (i, 0))],
        out_shape=[jax.ShapeDtypeStruct((z0.shape[0], D), jnp.float32),
                   jax.ShapeDtypeStruct((z0.shape[0], 8), jnp.float32)],
    )(z0, cd, ea, radw, eaw, w2, b2, attw, attb, cw1, cb1, cw2)


def _node_update(h, aggPa, aggPb, n1a, n1b, nb1, n2, nb2, w1a, w1b, b1):
    def body(h_ref, agga_ref, aggb_ref, n1a_r, n1b_r, nb1_r, n2_r, nb2_r,
             wa, wb, b1r, h_out, tr_out, tc_out):
        agg = (agga_ref[0] + agga_ref[1]) + (aggb_ref[0] + aggb_ref[1])
        o = jax.nn.silu(_mm(h_ref[...], n1a_r[...]) + _mm(agg, n1b_r[...])
                        + nb1_r[...])
        o = _mm(o, n2_r[...]) + nb2_r[...]
        hn = h_ref[...] + o
        h_out[...] = hn
        tr_out[...] = _mm(hn, wa[...]) + b1r[...]
        tc_out[...] = _mm(hn, wb[...])

    return pl.pallas_call(
        body, grid=(N // NB,),
        in_specs=[pl.BlockSpec((NB, D), lambda i: (i, 0)),
                  pl.BlockSpec((NC, NB, D), lambda i: (0, i, 0)),
                  pl.BlockSpec((NC, NB, D), lambda i: (0, i, 0)),
                  _full((D, D)), _full((D, D)), _full((1, D)), _full((D, D)),
                  _full((1, D)), _full((D, D)), _full((D, D)), _full((1, D))],
        out_specs=[pl.BlockSpec((NB, D), lambda i: (i, 0))] * 3,
        out_shape=[jax.ShapeDtypeStruct((N, D), jnp.float32)] * 3,
    )(h, aggPa, aggPb, n1a, n1b, nb1, n2, nb2, w1a, w1b, b1)


def _node_final(h, aggPa, aggPb, n1a, n1b, nb1, n2, nb2, ew, eb, hw1, hb1,
                hw2, hb2):
    def body(h_ref, agga_ref, aggb_ref, n1a_r, n1b_r, nb1_r, n2_r, nb2_r,
             ew_r, eb_r, hw1_r, hb1_r, hw2_r, hb2_r, out_ref):
        agg = (agga_ref[0] + agga_ref[1]) + (aggb_ref[0] + aggb_ref[1])
        o = jax.nn.silu(_mm(h_ref[...], n1a_r[...]) + _mm(agg, n1b_r[...])
                        + nb1_r[...])
        o = _mm(o, n2_r[...]) + nb2_r[...]
        hn = h_ref[...] + o
        he = _mm(hn, ew_r[...]) + eb_r[...]
        z = jax.nn.relu(_mm(he, hw1_r[...]) + hb1_r[...])
        out_ref[...] = _mm(z, hw2_r[...]) + hb2_r[...]

    return pl.pallas_call(
        body, grid=(N // NB,),
        in_specs=[pl.BlockSpec((NB, D), lambda i: (i, 0)),
                  pl.BlockSpec((NC, NB, D), lambda i: (0, i, 0)),
                  pl.BlockSpec((NC, NB, D), lambda i: (0, i, 0)),
                  _full((D, D)), _full((D, D)), _full((1, D)), _full((D, D)),
                  _full((1, D)), _full((D, D)), _full((1, D)), _full((D, D)),
                  _full((1, D)), _full((D, 1)), _full((1, 1))],
        out_specs=pl.BlockSpec((NB, 1), lambda i: (i, 0)),
        out_shape=jax.ShapeDtypeStruct((N, 1), jnp.float32),
    )(h, aggPa, aggPb, n1a, n1b, nb1, n2, nb2, ew, eb, hw1, hb1, hw2, hb2)


def _coord_update(coord, tcnPa, tcnPb):
    def body(c_ref, ta_ref, tb_ref, o_ref):
        t = ((ta_ref[0, :N, 0:4] + ta_ref[1, :N, 0:4])
             + (tb_ref[0, :N, 0:4] + tb_ref[1, :N, 0:4]))
        cnt = jnp.maximum(t[:, 3:4], 1.0)
        upd = t / cnt
        msk = lax.broadcasted_iota(jnp.int32, (N, 4), 1) < 3
        o_ref[...] = c_ref[...] + jnp.where(msk, upd, 0.0)

    return pl.pallas_call(
        body,
        in_specs=[_full((N, 4)), _full((NC, SPAD, 16)), _full((NC, SPAD, 16))],
        out_specs=_full((N, 4)),
        out_shape=jax.ShapeDtypeStruct((N, 4), jnp.float32),
    )(coord, tcnPa, tcnPb)


def kernel(x, pos, edge_index, edge_attr, emb_in_w, emb_in_b, edge_w1,
           edge_b1, edge_w2, edge_b2, att_w, att_b, coord_w1, coord_b1,
           coord_w2, node_w1, node_b1, node_w2, node_b2, emb_out_w,
           emb_out_b, head_w1, head_b1, head_w2, head_b2):
    row = edge_index[0]
    col = edge_index[1]
    coord_flat = jnp.pad(pos, ((0, 0), (0, 1))).reshape(-1)

    h, tr, tc = _embed_prep(x, emb_in_w, emb_in_b.reshape(1, D),
                            edge_w1[0][:D], edge_w1[0][D:2 * D],
                            edge_b1[0].reshape(1, D))
    E2 = E // 2
    rows = (row[:E2], row[E2:])
    cols = (col[:E2], col[E2:])
    eas = (edge_attr[:E2], edge_attr[E2:])
    logits = None
    for i in range(NLAYERS):
        radw = edge_w1[i][2 * D:2 * D + 1]
        eaw = edge_w1[i][2 * D + 1:]
        aggs, tcns = [], []
        zs = []
        for hh in range(2):
            z0, cdf = _gather_layer(tr, tc, coord_flat, rows[hh], cols[hh])
            zs.append((z0, cdf))
        for hh in range(2):
            z0, cdf = zs[hh]
            m, tq = _edge_mlp(z0, cdf.reshape(E2, 4), eas[hh],
                              radw, eaw,
                              edge_w2[i], edge_b2[i].reshape(1, D),
                              att_w[i], att_b[i].reshape(1, 1),
                              coord_w1[i], coord_b1[i].reshape(1, D),
                              coord_w2[i])
            aggP, tcnP = _scatter_layer(m, tq.reshape(-1), rows[hh])
            aggs.append(aggP)
            tcns.append(tcnP.reshape(NC, SPAD, 16))
        if i < NLAYERS - 1:
            coord_flat = _coord_update(coord_flat.reshape(N, 4),
                                       tcns[0], tcns[1]).reshape(-1)
            h, tr, tc = _node_update(h, aggs[0], aggs[1], node_w1[i][:D],
                                     node_w1[i][D:],
                                     node_b1[i].reshape(1, D), node_w2[i],
                                     node_b2[i].reshape(1, D),
                                     edge_w1[i + 1][:D],
                                     edge_w1[i + 1][D:2 * D],
                                     edge_b1[i + 1].reshape(1, D))
        else:
            logits = _node_final(h, aggs[0], aggs[1], node_w1[i][:D],
                                 node_w1[i][D:],
                                 node_b1[i].reshape(1, D), node_w2[i],
                                 node_b2[i].reshape(1, D),
                                 emb_out_w, emb_out_b.reshape(1, D),
                                 head_w1, head_b1.reshape(1, D),
                                 head_w2, head_b2.reshape(1, 1))
    return logits
